# baseline (speedup 1.0000x reference)
import functools

import jax
import jax.numpy as jnp
from jax import lax
from jax.experimental import pallas as pl
from jax.experimental.pallas import tpu as pltpu
from jax.experimental.pallas import tpu_sc as plsc

N_NODES = 50000
E_EDGES = 800000
C_IN = 16
EMB = 32

NT = 16
NC = 2
NP = 51200
STRIPE = NP // NT
K = 128
NCHUNK = E_EDGES // K
CH_PER_T = -(-NCHUNK // NT)

def _zero_rows(ref, ncols):
    rows = ref.shape[0]
    z = jnp.zeros((16,), jnp.float32)

    def body(r, _):
        for h in range(ncols // 16):
            ref[r, pl.ds(16 * h, 16)] = z
        return 0

    lax.fori_loop(0, rows, body, 0)


def _zero_flat(ref):
    z = jnp.zeros((16,), jnp.float32)

    def body(i, _):
        ref[pl.ds(16 * i, 16)] = z
        return 0

    lax.fori_loop(0, ref.shape[0] // 16, body, 0)


LOOPN = ((CH_PER_T + 3) // 4) * 4


def _sc_pool_values_body(dv, ei, acc_out, cnt_out,
                         ib0, ib1, ib2, ib3, vb0, vb1, vb2, vb3,
                         ones_v, zrow, zflat,
                         si0, si1, si2, si3, ss0, ss1, ss2, ss3,
                         acc_sh, cnt_sh):
    cid = lax.axis_index("c")
    sid = lax.axis_index("s")
    ib = [ib0, ib1, ib2, ib3]
    vb = [vb0, vb1, vb2, vb3]
    semi = [si0, si1, si2, si3]
    sems = [ss0, ss1, ss2, ss3]

    _zero_rows(zrow, C_IN)
    _zero_flat(zflat)

    one = jnp.ones((16,), jnp.float32)

    def setones(i, _):
        ones_v[pl.ds(16 * i, 16)] = one
        return 0

    lax.fori_loop(0, K // 16, setones, 0)

    def zinit(j, _):
        base = sid * STRIPE + j * K
        pltpu.sync_copy(zrow, acc_sh.at[pl.ds(base, K), :])
        pltpu.sync_copy(zflat, cnt_sh.at[pl.ds(base, K)])
        return 0

    lax.fori_loop(0, STRIPE // K, zinit, 0)
    plsc.subcore_barrier()

    def valid(i):
        return jnp.logical_and(i >= 0, i * NT + sid < NCHUNK)

    def fire_loads(i, s):
        chunk = i * NT + sid
        pltpu.async_copy(ei.at[cid, pl.ds(chunk * K, K)], ib[s], semi[s])
        pltpu.async_copy(dv.at[pl.ds(chunk * K, K), :], vb[s], semi[s])

    def wait_loads(s):
        pltpu.make_async_copy(ei.at[0, pl.ds(0, K)], ib[s], semi[s]).wait()
        pltpu.make_async_copy(dv.at[pl.ds(0, K), :], vb[s], semi[s]).wait()

    def fire_scatters(s):
        pltpu.async_copy(vb[s], acc_sh.at[ib[s]], sems[s], add=True)
        pltpu.async_copy(ones_v, cnt_sh.at[ib[s]], sems[s], add=True)

    def wait_scatters(s):
        pltpu.make_async_copy(vb[s], acc_sh.at[ib[s]], sems[s]).wait()
        pltpu.make_async_copy(ones_v, cnt_sh.at[ib[s]], sems[s]).wait()

    @pl.when(valid(0))
    def _():
        fire_loads(0, 0)

    def quad(j, _):
        for u in range(4):
            i = j * 4 + u

            @pl.when(valid(i - 2))
            def _():
                wait_scatters((u + 2) & 3)

            @pl.when(valid(i + 1))
            def _():
                fire_loads(i + 1, (u + 1) & 3)

            @pl.when(valid(i))
            def _():
                wait_loads(u)
                fire_scatters(u)

        return 0

    lax.fori_loop(0, LOOPN // 4, quad, 0)

    @pl.when(valid(LOOPN - 2))
    def _():
        wait_scatters((LOOPN - 2) & 3)

    @pl.when(valid(LOOPN - 1))
    def _():
        wait_scatters((LOOPN - 1) & 3)

    plsc.subcore_barrier()

    rows = pl.ds(sid * STRIPE, STRIPE)
    pltpu.sync_copy(acc_sh.at[rows, :], acc_out.at[cid, rows, :])
    pltpu.sync_copy(cnt_sh.at[rows], cnt_out.at[cid, rows])


def _sc_edge_stage_body(h, ei, acc_out,
                        ib0, ib1, ib2, ib3, hr0, hr1, hr2, hr3,
                        hc0, hc1, zrow,
                        si0, si1, si2, si3, sg0, sg1, sg2, sg3,
                        ss0, ss1, ss2, ss3, acc_sh):
    cid = lax.axis_index("c")
    sid = lax.axis_index("s")
    ib = [ib0, ib1, ib2, ib3]
    hr = [hr0, hr1, hr2, hr3]
    hc = [hc0, hc1]
    semi = [si0, si1, si2, si3]
    semg = [sg0, sg1, sg2, sg3]
    sems = [ss0, ss1, ss2, ss3]

    _zero_rows(zrow, EMB)

    def zinit(j, _):
        pltpu.sync_copy(zrow, acc_sh.at[pl.ds(sid * STRIPE + j * 64, 64), :])
        return 0

    lax.fori_loop(0, STRIPE // 64, zinit, 0)
    plsc.subcore_barrier()

    def valid(i):
        return jnp.logical_and(i >= 0, i * NT + sid < NCHUNK)

    def fire_idx(i, s):
        base = (i * NT + sid) * K
        pltpu.async_copy(ei.at[0, pl.ds(base, K)], ib[s].at[0], semi[s])
        pltpu.async_copy(ei.at[1, pl.ds(base, K)], ib[s].at[1], semi[s])

    def wait_idx(s):
        pltpu.make_async_copy(ei.at[0, pl.ds(0, K)], ib[s].at[0], semi[s]).wait()
        pltpu.make_async_copy(ei.at[1, pl.ds(0, K)], ib[s].at[1], semi[s]).wait()

    def fire_gathers(s, p):
        pltpu.async_copy(h.at[ib[s].at[0]], hr[s], semg[s])
        pltpu.async_copy(h.at[ib[s].at[1]], hc[p], semg[s])

    def wait_gathers(s, p):
        pltpu.make_async_copy(h.at[ib[s].at[0]], hr[s], semg[s]).wait()
        pltpu.make_async_copy(h.at[ib[s].at[1]], hc[p], semg[s]).wait()

    def fire_scatter(s):
        pltpu.async_copy(hr[s], acc_sh.at[ib[s].at[cid]], sems[s], add=True)

    def wait_scatter(s):
        pltpu.make_async_copy(hr[s], acc_sh.at[ib[s].at[cid]], sems[s]).wait()

    @pl.when(valid(0))
    def _():
        fire_idx(0, 0)

    @pl.when(valid(1))
    def _():
        fire_idx(1, 1)

    @pl.when(valid(0))
    def _():
        wait_idx(0)
        fire_gathers(0, 0)

    def quad(j, _):
        for u in range(4):
            i = j * 4 + u

            @pl.when(valid(i - 2))
            def _():
                wait_scatter((u + 2) & 3)

            @pl.when(valid(i + 2))
            def _():
                fire_idx(i + 2, (u + 2) & 3)

            @pl.when(valid(i + 1))
            def _():
                wait_idx((u + 1) & 3)
                fire_gathers((u + 1) & 3, (u + 1) & 1)

            @pl.when(valid(i))
            def _():
                wait_gathers(u, u & 1)

                def compute(e4, _):
                    for v in range(4):
                        e = e4 * 4 + v
                        for half in range(EMB // 16):
                            sl = pl.ds(16 * half, 16)
                            hr[u][e, sl] = jnp.maximum(
                                hr[u][e, sl] + hc[u & 1][e, sl], 0.0)
                    return 0

                lax.fori_loop(0, K // 4, compute, 0)
                fire_scatter(u)

        return 0

    lax.fori_loop(0, LOOPN // 4, quad, 0)

    @pl.when(valid(LOOPN - 2))
    def _():
        wait_scatter((LOOPN - 2) & 3)

    @pl.when(valid(LOOPN - 1))
    def _():
        wait_scatter((LOOPN - 1) & 3)

    plsc.subcore_barrier()

    rows = pl.ds(sid * STRIPE, STRIPE)
    pltpu.sync_copy(acc_sh.at[rows, :], acc_out.at[cid, rows, :])


@functools.lru_cache(maxsize=1)
def _sc_kernels():
    mesh = plsc.VectorSubcoreMesh(core_axis_name="c", subcore_axis_name="s")
    params = pltpu.CompilerParams(use_tc_tiling_on_sc=False)
    pool_values = pl.kernel(
        _sc_pool_values_body,
        out_type=[
            jax.ShapeDtypeStruct((NC, NP, C_IN), jnp.float32),
            jax.ShapeDtypeStruct((NC, NP), jnp.float32),
        ],
        mesh=mesh,
        scratch_types=(
            [pltpu.VMEM((K,), jnp.int32)] * 4
            + [pltpu.VMEM((K, C_IN), jnp.float32)] * 4
            + [pltpu.VMEM((K,), jnp.float32),
               pltpu.VMEM((K, C_IN), jnp.float32),
               pltpu.VMEM((K,), jnp.float32)]
            + [pltpu.SemaphoreType.DMA] * 8
            + [pltpu.MemorySpace.VMEM_SHARED((NP, C_IN), jnp.float32),
               pltpu.MemorySpace.VMEM_SHARED((NP,), jnp.float32)]
        ),
        compiler_params=params,
    )
    edge_stage = pl.kernel(
        _sc_edge_stage_body,
        out_type=jax.ShapeDtypeStruct((NC, NP, EMB), jnp.float32),
        mesh=mesh,
        scratch_types=(
            [pltpu.VMEM((2, K), jnp.int32)] * 4
            + [pltpu.VMEM((K, EMB), jnp.float32)] * 6
            + [pltpu.VMEM((64, EMB), jnp.float32)]
            + [pltpu.SemaphoreType.DMA] * 12
            + [pltpu.MemorySpace.VMEM_SHARED((NP, EMB), jnp.float32)]
        ),
        compiler_params=params,
    )
    return pool_values, edge_stage


BLK = 3200
NB = NP // BLK


def _mean_of(acc_r, acc_c, cnt_r, cnt_c):
    cr = cnt_r[0]
    cc = cnt_c[0]
    mean = acc_r[0] / jnp.maximum(cr, 1.0) + acc_c[0] / jnp.maximum(cc, 1.0)
    gate = (cr > 0.0).astype(jnp.float32) + (cc > 0.0).astype(jnp.float32)
    return mean, gate


def _layer_body(with_emb):
    def body(*refs):
        if with_emb:
            acc_r, acc_c, cnt_r, cnt_c, emb0, wp, bp, wb, bb, h_out, xs, st = refs
        else:
            acc_r, acc_c, cnt_r, cnt_c, wp, bp, wb, bb, h_out, xs, st = refs
        ph = pl.program_id(0)
        i = pl.program_id(1)

        @pl.when(ph == 0)
        def _():
            mean, gate = _mean_of(acc_r, acc_c, cnt_r, cnt_c)
            pooled = (jnp.dot(mean, wp[:], preferred_element_type=jnp.float32)
                      + gate * bp[:])
            if with_emb:
                rid = i * BLK + lax.broadcasted_iota(jnp.int32, (BLK, 1), 0)
                pooled = pooled + jnp.where(rid < N_NODES, emb0[:], 0.0)
            x = jnp.maximum(pooled, 0.0)
            xs[pl.ds(i * BLK, BLK), :] = x
            h_out[:] = x
            part = jnp.concatenate(
                [jnp.sum(x, axis=0, keepdims=True),
                 jnp.sum(x * x, axis=0, keepdims=True)], axis=0)

            @pl.when(i == 0)
            def _():
                st[:] = part

            @pl.when(i != 0)
            def _():
                st[:] = st[:] + part

        @pl.when(ph == 1)
        def _():
            m = st[0:1, :] / N_NODES
            v = st[1:2, :] / N_NODES - m * m
            y = (xs[pl.ds(i * BLK, BLK), :] - m) * lax.rsqrt(v + 1e-5)
            h_out[:] = jnp.dot(y, wb[:], preferred_element_type=jnp.float32) + bb[:]

    return body


def _densef_body(acc_r, acc_c, cnt_r, cnt_c, wp, bp, out):
    mean, gate = _mean_of(acc_r, acc_c, cnt_r, cnt_c)
    out[:] = jnp.dot(mean, wp[:], preferred_element_type=jnp.float32) + gate * bp[:]


def _half2(core, c):
    return pl.BlockSpec((1, BLK, c), lambda p, i, _c=core: (_c, i, 0))


def _rows2(c):
    return pl.BlockSpec((BLK, c), lambda p, i: (i, 0))


def _full2(r, c):
    return pl.BlockSpec((r, c), lambda p, i: (0, 0))


def _dense_layer(acc, cnt3, emb0, wp, bp, wb, bb):
    cin = acc.shape[2]
    with_emb = emb0 is not None
    specs = [_half2(0, cin), _half2(1, cin), _half2(0, 1), _half2(1, 1)]
    args = [acc, acc, cnt3, cnt3]
    if with_emb:
        specs.append(_rows2(EMB))
        args.append(emb0)
    specs += [_full2(cin, EMB), _full2(1, EMB), _full2(EMB, EMB), _full2(1, EMB)]
    args += [wp, bp, wb, bb]
    return pl.pallas_call(
        _layer_body(with_emb),
        grid=(2, NB),
        in_specs=specs,
        out_specs=_rows2(EMB),
        out_shape=jax.ShapeDtypeStruct((NP, EMB), jnp.float32),
        scratch_shapes=[pltpu.VMEM((NP, EMB), jnp.float32),
                        pltpu.VMEM((2, EMB), jnp.float32)],
    )(*args)


def _densef(acc, cnt3, wp, bp):
    return pl.pallas_call(
        _densef_body,
        grid=(1, NB),
        in_specs=[_half2(0, EMB), _half2(1, EMB), _half2(0, 1), _half2(1, 1),
                  _full2(EMB, 1), _full2(1, 1)],
        out_specs=_rows2(1),
        out_shape=jax.ShapeDtypeStruct((NP, 1), jnp.float32),
    )(acc, acc, cnt3, cnt3, wp, bp)


def kernel(data_values, data_embedding, edge_index, W_pool0, b_pool0,
           W_pool1, b_pool1, W_pool2, b_pool2, W_bc0, b_bc0, W_bc1, b_bc1):
    ei = edge_index.astype(jnp.int32)
    _sc_pool_values, _sc_edge_stage = _sc_kernels()

    acc0, cnt = _sc_pool_values(data_values, ei)
    cnt3 = cnt.reshape(NC, NP, 1)

    h0 = _dense_layer(acc0, cnt3, data_embedding, W_pool0,
                      b_pool0.reshape(1, EMB), W_bc0, b_bc0.reshape(1, EMB))
    acc1 = _sc_edge_stage(h0, ei)
    h1 = _dense_layer(acc1, cnt3, None, W_pool1,
                      b_pool1.reshape(1, EMB), W_bc1, b_bc1.reshape(1, EMB))
    acc2 = _sc_edge_stage(h1, ei)
    out = _densef(acc2, cnt3, W_pool2, b_pool2.reshape(1, 1))
    return out[:N_NODES]


# speedup vs baseline: 1.0336x; 1.0336x over previous
import functools

import jax
import jax.numpy as jnp
from jax import lax
from jax.experimental import pallas as pl
from jax.experimental.pallas import tpu as pltpu
from jax.experimental.pallas import tpu_sc as plsc

N_NODES = 50000
E_EDGES = 800000
C_IN = 16
EMB = 32

NT = 16
NC = 2
NP = 51200
STRIPE = NP // NT
K = 128
NCHUNK = E_EDGES // K
CH_PER_T = -(-NCHUNK // NT)

def _zero_rows(ref, ncols):
    rows = ref.shape[0]
    z = jnp.zeros((16,), jnp.float32)

    def body(r, _):
        for h in range(ncols // 16):
            ref[r, pl.ds(16 * h, 16)] = z
        return 0

    lax.fori_loop(0, rows, body, 0)


def _zero_flat(ref):
    z = jnp.zeros((16,), jnp.float32)

    def body(i, _):
        ref[pl.ds(16 * i, 16)] = z
        return 0

    lax.fori_loop(0, ref.shape[0] // 16, body, 0)


LOOPN = ((CH_PER_T + 3) // 4) * 4


def _sc_pool_values_body(dv, ei, acc_out, cnt_out,
                         ib0, ib1, ib2, ib3, vb0, vb1, vb2, vb3,
                         ones_v, zrow, zflat,
                         si0, si1, si2, si3, ss0, ss1, ss2, ss3,
                         acc_sh, cnt_sh):
    cid = lax.axis_index("c")
    sid = lax.axis_index("s")
    ib = [ib0, ib1, ib2, ib3]
    vb = [vb0, vb1, vb2, vb3]
    semi = [si0, si1, si2, si3]
    sems = [ss0, ss1, ss2, ss3]

    _zero_rows(zrow, C_IN)
    _zero_flat(zflat)

    one = jnp.ones((16,), jnp.float32)

    def setones(i, _):
        ones_v[pl.ds(16 * i, 16)] = one
        return 0

    lax.fori_loop(0, K // 16, setones, 0)

    def zinit(j, _):
        base = sid * STRIPE + j * K
        pltpu.sync_copy(zrow, acc_sh.at[pl.ds(base, K), :])
        pltpu.sync_copy(zflat, cnt_sh.at[pl.ds(base, K)])
        return 0

    lax.fori_loop(0, STRIPE // K, zinit, 0)
    plsc.subcore_barrier()

    def valid(i):
        return jnp.logical_and(i >= 0, i * NT + sid < NCHUNK)

    def fire_loads(i, s):
        chunk = i * NT + sid
        pltpu.async_copy(ei.at[cid, pl.ds(chunk * K, K)], ib[s], semi[s])
        pltpu.async_copy(dv.at[pl.ds(chunk * K, K), :], vb[s], semi[s])

    def wait_loads(s):
        pltpu.make_async_copy(ei.at[0, pl.ds(0, K)], ib[s], semi[s]).wait()
        pltpu.make_async_copy(dv.at[pl.ds(0, K), :], vb[s], semi[s]).wait()

    def fire_scatters(s):
        pltpu.async_copy(vb[s], acc_sh.at[ib[s]], sems[s], add=True)
        pltpu.async_copy(ones_v, cnt_sh.at[ib[s]], sems[s], add=True)

    def wait_scatters(s):
        pltpu.make_async_copy(vb[s], acc_sh.at[ib[s]], sems[s]).wait()
        pltpu.make_async_copy(ones_v, cnt_sh.at[ib[s]], sems[s]).wait()

    @pl.when(valid(0))
    def _():
        fire_loads(0, 0)

    def quad(j, _):
        for u in range(4):
            i = j * 4 + u

            @pl.when(valid(i - 2))
            def _():
                wait_scatters((u + 2) & 3)

            @pl.when(valid(i + 1))
            def _():
                fire_loads(i + 1, (u + 1) & 3)

            @pl.when(valid(i))
            def _():
                wait_loads(u)
                fire_scatters(u)

        return 0

    lax.fori_loop(0, LOOPN // 4, quad, 0)

    @pl.when(valid(LOOPN - 2))
    def _():
        wait_scatters((LOOPN - 2) & 3)

    @pl.when(valid(LOOPN - 1))
    def _():
        wait_scatters((LOOPN - 1) & 3)

    plsc.subcore_barrier()

    rows = pl.ds(sid * STRIPE, STRIPE)
    pltpu.sync_copy(acc_sh.at[rows, :], acc_out.at[cid, rows, :])
    pltpu.sync_copy(cnt_sh.at[rows], cnt_out.at[cid, rows])


def _sc_edge_stage_body(h, ei, acc_out,
                        ib0, ib1, ib2, ib3, hr0, hr1, hr2, hr3,
                        hc0, hc1, zrow,
                        si0, si1, si2, si3, sg0, sg1, sg2, sg3,
                        ss0, ss1, ss2, ss3, acc_sh):
    cid = lax.axis_index("c")
    sid = lax.axis_index("s")
    ib = [ib0, ib1, ib2, ib3]
    hr = [hr0, hr1, hr2, hr3]
    hc = [hc0, hc1]
    semi = [si0, si1, si2, si3]
    semg = [sg0, sg1, sg2, sg3]
    sems = [ss0, ss1, ss2, ss3]

    _zero_rows(zrow, EMB)

    def zinit(j, _):
        pltpu.sync_copy(zrow, acc_sh.at[pl.ds(sid * STRIPE + j * 64, 64), :])
        return 0

    lax.fori_loop(0, STRIPE // 64, zinit, 0)
    plsc.subcore_barrier()

    def valid(i):
        return jnp.logical_and(i >= 0, i * NT + sid < NCHUNK)

    def fire_idx(i, s):
        base = (i * NT + sid) * K
        pltpu.async_copy(ei.at[0, pl.ds(base, K)], ib[s].at[0], semi[s])
        pltpu.async_copy(ei.at[1, pl.ds(base, K)], ib[s].at[1], semi[s])

    def wait_idx(s):
        pltpu.make_async_copy(ei.at[0, pl.ds(0, K)], ib[s].at[0], semi[s]).wait()
        pltpu.make_async_copy(ei.at[1, pl.ds(0, K)], ib[s].at[1], semi[s]).wait()

    def fire_gathers(s, p):
        pltpu.async_copy(h.at[ib[s].at[0]], hr[s], semg[s])
        pltpu.async_copy(h.at[ib[s].at[1]], hc[p], semg[s])

    def wait_gathers(s, p):
        pltpu.make_async_copy(h.at[ib[s].at[0]], hr[s], semg[s]).wait()
        pltpu.make_async_copy(h.at[ib[s].at[1]], hc[p], semg[s]).wait()

    def fire_scatter(s):
        pltpu.async_copy(hr[s], acc_sh.at[ib[s].at[cid]], sems[s], add=True)

    def wait_scatter(s):
        pltpu.make_async_copy(hr[s], acc_sh.at[ib[s].at[cid]], sems[s]).wait()

    @pl.when(valid(0))
    def _():
        fire_idx(0, 0)

    @pl.when(valid(1))
    def _():
        fire_idx(1, 1)

    @pl.when(valid(0))
    def _():
        wait_idx(0)
        fire_gathers(0, 0)

    def quad(j, _):
        for u in range(4):
            i = j * 4 + u

            @pl.when(valid(i - 2))
            def _():
                wait_scatter((u + 2) & 3)

            @pl.when(valid(i + 2))
            def _():
                fire_idx(i + 2, (u + 2) & 3)

            @pl.when(valid(i + 1))
            def _():
                wait_idx((u + 1) & 3)
                fire_gathers((u + 1) & 3, (u + 1) & 1)

            @pl.when(valid(i))
            def _():
                wait_gathers(u, u & 1)

                def compute(e4, _):
                    for v in range(4):
                        e = e4 * 4 + v
                        for half in range(EMB // 16):
                            sl = pl.ds(16 * half, 16)
                            hr[u][e, sl] = jnp.maximum(
                                hr[u][e, sl] + hc[u & 1][e, sl], 0.0)
                    return 0

                lax.fori_loop(0, K // 4, compute, 0)
                fire_scatter(u)

        return 0

    lax.fori_loop(0, LOOPN // 4, quad, 0)

    @pl.when(valid(LOOPN - 2))
    def _():
        wait_scatter((LOOPN - 2) & 3)

    @pl.when(valid(LOOPN - 1))
    def _():
        wait_scatter((LOOPN - 1) & 3)

    plsc.subcore_barrier()

    rows = pl.ds(sid * STRIPE, STRIPE)
    pltpu.sync_copy(acc_sh.at[rows, :], acc_out.at[cid, rows, :])


@functools.lru_cache(maxsize=1)
def _sc_kernels():
    mesh = plsc.VectorSubcoreMesh(core_axis_name="c", subcore_axis_name="s")
    params = pltpu.CompilerParams(use_tc_tiling_on_sc=False)
    pool_values = pl.kernel(
        _sc_pool_values_body,
        out_type=[
            jax.ShapeDtypeStruct((NC, NP, C_IN), jnp.float32),
            jax.ShapeDtypeStruct((NC, NP), jnp.float32),
        ],
        mesh=mesh,
        scratch_types=(
            [pltpu.VMEM((K,), jnp.int32)] * 4
            + [pltpu.VMEM((K, C_IN), jnp.float32)] * 4
            + [pltpu.VMEM((K,), jnp.float32),
               pltpu.VMEM((K, C_IN), jnp.float32),
               pltpu.VMEM((K,), jnp.float32)]
            + [pltpu.SemaphoreType.DMA] * 8
            + [pltpu.MemorySpace.VMEM_SHARED((NP, C_IN), jnp.float32),
               pltpu.MemorySpace.VMEM_SHARED((NP,), jnp.float32)]
        ),
        compiler_params=params,
    )
    edge_stage = pl.kernel(
        _sc_edge_stage_body,
        out_type=jax.ShapeDtypeStruct((NC, NP, EMB), jnp.float32),
        mesh=mesh,
        scratch_types=(
            [pltpu.VMEM((2, K), jnp.int32)] * 4
            + [pltpu.VMEM((K, EMB), jnp.float32)] * 6
            + [pltpu.VMEM((64, EMB), jnp.float32)]
            + [pltpu.SemaphoreType.DMA] * 12
            + [pltpu.MemorySpace.VMEM_SHARED((NP, EMB), jnp.float32)]
        ),
        compiler_params=params,
    )
    return pool_values, edge_stage


BLK = 6400
NB = NP // BLK


def _mean_of(acc_r, acc_c, cnt_r, cnt_c):
    cr = cnt_r[0]
    cc = cnt_c[0]
    mean = acc_r[0] / jnp.maximum(cr, 1.0) + acc_c[0] / jnp.maximum(cc, 1.0)
    gate = (cr > 0.0).astype(jnp.float32) + (cc > 0.0).astype(jnp.float32)
    return mean, gate


def _pre_body(with_emb):
    def body(*refs):
        if with_emb:
            acc_r, acc_c, cnt_r, cnt_c, emb0, wp, bp, x_out, st_out = refs
        else:
            acc_r, acc_c, cnt_r, cnt_c, wp, bp, x_out, st_out = refs
        mean, gate = _mean_of(acc_r, acc_c, cnt_r, cnt_c)
        pooled = (jnp.dot(mean, wp[:], preferred_element_type=jnp.float32)
                  + gate * bp[:])
        if with_emb:
            rid = (pl.program_id(0) * BLK
                   + lax.broadcasted_iota(jnp.int32, (BLK, 1), 0))
            pooled = pooled + jnp.where(rid < N_NODES, emb0[:], 0.0)
        x = jnp.maximum(pooled, 0.0)
        x_out[:] = x
        part = jnp.concatenate(
            [jnp.sum(x, axis=0, keepdims=True),
             jnp.sum(x * x, axis=0, keepdims=True)], axis=0)

        @pl.when(pl.program_id(0) == 0)
        def _():
            st_out[:] = part

        @pl.when(pl.program_id(0) != 0)
        def _():
            st_out[:] = st_out[:] + part

    return body


def _post_body(x, st, wb, bb, out):
    m = st[0:1, :] / N_NODES
    v = st[1:2, :] / N_NODES - m * m
    y = (x[:] - m) * lax.rsqrt(v + 1e-5)
    out[:] = jnp.dot(y, wb[:], preferred_element_type=jnp.float32) + bb[:]


def _densef_body(acc_r, acc_c, cnt_r, cnt_c, wp, bp, out):
    mean, gate = _mean_of(acc_r, acc_c, cnt_r, cnt_c)
    out[:] = jnp.dot(mean, wp[:], preferred_element_type=jnp.float32) + gate * bp[:]


def _half(core, c):
    return pl.BlockSpec((1, BLK, c), lambda i, _c=core: (_c, i, 0))


def _rows(c):
    return pl.BlockSpec((BLK, c), lambda i: (i, 0))


def _full(r, c):
    return pl.BlockSpec((r, c), lambda i: (0, 0))


def _dense_pre(acc, cnt3, emb0, wp, bp):
    cin = acc.shape[2]
    with_emb = emb0 is not None
    specs = [_half(0, cin), _half(1, cin), _half(0, 1), _half(1, 1)]
    args = [acc, acc, cnt3, cnt3]
    if with_emb:
        specs.append(_rows(EMB))
        args.append(emb0)
    specs += [_full(cin, EMB), _full(1, EMB)]
    args += [wp, bp]
    return pl.pallas_call(
        _pre_body(with_emb),
        grid=(NB,),
        in_specs=specs,
        out_specs=[_rows(EMB), _full(2, EMB)],
        out_shape=[jax.ShapeDtypeStruct((NP, EMB), jnp.float32),
                   jax.ShapeDtypeStruct((2, EMB), jnp.float32)],
    )(*args)


def _dense_post(x, st, wb, bb):
    return pl.pallas_call(
        _post_body,
        grid=(NB,),
        in_specs=[_rows(EMB), _full(2, EMB), _full(EMB, EMB), _full(1, EMB)],
        out_specs=_rows(EMB),
        out_shape=jax.ShapeDtypeStruct((NP, EMB), jnp.float32),
    )(x, st, wb, bb)


def _densef(acc, cnt3, wp, bp):
    return pl.pallas_call(
        _densef_body,
        grid=(NB,),
        in_specs=[_half(0, EMB), _half(1, EMB), _half(0, 1), _half(1, 1),
                  _full(EMB, 1), _full(1, 1)],
        out_specs=_rows(1),
        out_shape=jax.ShapeDtypeStruct((NP, 1), jnp.float32),
    )(acc, acc, cnt3, cnt3, wp, bp)


def kernel(data_values, data_embedding, edge_index, W_pool0, b_pool0,
           W_pool1, b_pool1, W_pool2, b_pool2, W_bc0, b_bc0, W_bc1, b_bc1):
    ei = edge_index.astype(jnp.int32)
    _sc_pool_values, _sc_edge_stage = _sc_kernels()

    acc0, cnt = _sc_pool_values(data_values, ei)
    cnt3 = cnt.reshape(NC, NP, 1)

    x0, st0 = _dense_pre(acc0, cnt3, data_embedding,
                         W_pool0, b_pool0.reshape(1, EMB))
    h0 = _dense_post(x0, st0, W_bc0, b_bc0.reshape(1, EMB))
    acc1 = _sc_edge_stage(h0, ei)
    x1, st1 = _dense_pre(acc1, cnt3, None, W_pool1, b_pool1.reshape(1, EMB))
    h1 = _dense_post(x1, st1, W_bc1, b_bc1.reshape(1, EMB))
    acc2 = _sc_edge_stage(h1, ei)
    out = _densef(acc2, cnt3, W_pool2, b_pool2.reshape(1, 1))
    return out[:N_NODES]


# speedup vs baseline: 1.0358x; 1.0021x over previous
import functools

import jax
import jax.numpy as jnp
from jax import lax
from jax.experimental import pallas as pl
from jax.experimental.pallas import tpu as pltpu
from jax.experimental.pallas import tpu_sc as plsc

N_NODES = 50000
E_EDGES = 800000
C_IN = 16
EMB = 32

NT = 16
NC = 2
NP = 51200
STRIPE = NP // NT
K = 128
NCHUNK = E_EDGES // K
CH_PER_T = -(-NCHUNK // NT)

def _zero_rows(ref, ncols):
    rows = ref.shape[0]
    z = jnp.zeros((16,), jnp.float32)

    def body(r, _):
        for h in range(ncols // 16):
            ref[r, pl.ds(16 * h, 16)] = z
        return 0

    lax.fori_loop(0, rows, body, 0)


def _zero_flat(ref):
    z = jnp.zeros((16,), jnp.float32)

    def body(i, _):
        ref[pl.ds(16 * i, 16)] = z
        return 0

    lax.fori_loop(0, ref.shape[0] // 16, body, 0)


LOOPN = ((CH_PER_T + 3) // 4) * 4


def _sc_pool_values_body(dv, ei, acc_out, cnt_out,
                         ib0, ib1, ib2, ib3, vb0, vb1, vb2, vb3,
                         ones_v, zrow, zflat,
                         si0, si1, si2, si3, ss0, ss1, ss2, ss3,
                         acc_sh, cnt_sh):
    cid = lax.axis_index("c")
    sid = lax.axis_index("s")
    ib = [ib0, ib1, ib2, ib3]
    vb = [vb0, vb1, vb2, vb3]
    semi = [si0, si1, si2, si3]
    sems = [ss0, ss1, ss2, ss3]

    _zero_rows(zrow, C_IN)
    _zero_flat(zflat)

    one = jnp.ones((16,), jnp.float32)

    def setones(i, _):
        ones_v[pl.ds(16 * i, 16)] = one
        return 0

    lax.fori_loop(0, K // 16, setones, 0)

    def zinit(j, _):
        base = sid * STRIPE + j * K
        pltpu.sync_copy(zrow, acc_sh.at[pl.ds(base, K), :])
        pltpu.sync_copy(zflat, cnt_sh.at[pl.ds(base, K)])
        return 0

    lax.fori_loop(0, STRIPE // K, zinit, 0)
    plsc.subcore_barrier()

    def valid(i):
        return jnp.logical_and(i >= 0, i * NT + sid < NCHUNK)

    def fire_loads(i, s):
        chunk = i * NT + sid
        pltpu.async_copy(ei.at[cid, pl.ds(chunk * K, K)], ib[s], semi[s])
        pltpu.async_copy(dv.at[pl.ds(chunk * K, K), :], vb[s], semi[s])

    def wait_loads(s):
        pltpu.make_async_copy(ei.at[0, pl.ds(0, K)], ib[s], semi[s]).wait()
        pltpu.make_async_copy(dv.at[pl.ds(0, K), :], vb[s], semi[s]).wait()

    def fire_scatters(s):
        pltpu.async_copy(vb[s], acc_sh.at[ib[s]], sems[s], add=True)
        pltpu.async_copy(ones_v, cnt_sh.at[ib[s]], sems[s], add=True)

    def wait_scatters(s):
        pltpu.make_async_copy(vb[s], acc_sh.at[ib[s]], sems[s]).wait()
        pltpu.make_async_copy(ones_v, cnt_sh.at[ib[s]], sems[s]).wait()

    @pl.when(valid(0))
    def _():
        fire_loads(0, 0)

    def quad(j, _):
        for u in range(4):
            i = j * 4 + u

            @pl.when(valid(i - 2))
            def _():
                wait_scatters((u + 2) & 3)

            @pl.when(valid(i + 1))
            def _():
                fire_loads(i + 1, (u + 1) & 3)

            @pl.when(valid(i))
            def _():
                wait_loads(u)
                fire_scatters(u)

        return 0

    lax.fori_loop(0, LOOPN // 4, quad, 0)

    @pl.when(valid(LOOPN - 2))
    def _():
        wait_scatters((LOOPN - 2) & 3)

    @pl.when(valid(LOOPN - 1))
    def _():
        wait_scatters((LOOPN - 1) & 3)

    plsc.subcore_barrier()

    rows = pl.ds(sid * STRIPE, STRIPE)
    pltpu.sync_copy(acc_sh.at[rows, :], acc_out.at[cid, rows, :])
    pltpu.sync_copy(cnt_sh.at[rows], cnt_out.at[cid, rows])


def _sc_edge_stage_body(h, ei, acc_out,
                        ib0, ib1, ib2, ib3, hr0, hr1, hr2, hr3,
                        hc0, hc1, zrow,
                        si0, si1, si2, si3, sg0, sg1, sg2, sg3,
                        ss0, ss1, ss2, ss3, acc_sh):
    cid = lax.axis_index("c")
    sid = lax.axis_index("s")
    ib = [ib0, ib1, ib2, ib3]
    hr = [hr0, hr1, hr2, hr3]
    hc = [hc0, hc1]
    semi = [si0, si1, si2, si3]
    semg = [sg0, sg1, sg2, sg3]
    sems = [ss0, ss1, ss2, ss3]

    _zero_rows(zrow, EMB)

    def zinit(j, _):
        pltpu.sync_copy(zrow, acc_sh.at[pl.ds(sid * STRIPE + j * 64, 64), :])
        return 0

    lax.fori_loop(0, STRIPE // 64, zinit, 0)
    plsc.subcore_barrier()

    def valid(i):
        return jnp.logical_and(i >= 0, i * NT + sid < NCHUNK)

    def fire_idx(i, s):
        base = (i * NT + sid) * K
        pltpu.async_copy(ei.at[:, pl.ds(base, K)], ib[s], semi[s])

    def wait_idx(s):
        pltpu.make_async_copy(ei.at[:, pl.ds(0, K)], ib[s], semi[s]).wait()

    def fire_gathers(s, p):
        pltpu.async_copy(h.at[ib[s].at[0]], hr[s], semg[s])
        pltpu.async_copy(h.at[ib[s].at[1]], hc[p], semg[s])

    def wait_gathers(s, p):
        pltpu.make_async_copy(h.at[ib[s].at[0]], hr[s], semg[s]).wait()
        pltpu.make_async_copy(h.at[ib[s].at[1]], hc[p], semg[s]).wait()

    def fire_scatter(s):
        pltpu.async_copy(hr[s], acc_sh.at[ib[s].at[cid]], sems[s], add=True)

    def wait_scatter(s):
        pltpu.make_async_copy(hr[s], acc_sh.at[ib[s].at[cid]], sems[s]).wait()

    @pl.when(valid(0))
    def _():
        fire_idx(0, 0)

    @pl.when(valid(1))
    def _():
        fire_idx(1, 1)

    @pl.when(valid(0))
    def _():
        wait_idx(0)
        fire_gathers(0, 0)

    def quad(j, _):
        for u in range(4):
            i = j * 4 + u

            @pl.when(valid(i - 2))
            def _():
                wait_scatter((u + 2) & 3)

            @pl.when(valid(i + 2))
            def _():
                fire_idx(i + 2, (u + 2) & 3)

            @pl.when(valid(i + 1))
            def _():
                wait_idx((u + 1) & 3)
                fire_gathers((u + 1) & 3, (u + 1) & 1)

            @pl.when(valid(i))
            def _():
                wait_gathers(u, u & 1)

                def compute(e4, _):
                    for v in range(4):
                        e = e4 * 4 + v
                        for half in range(EMB // 16):
                            sl = pl.ds(16 * half, 16)
                            hr[u][e, sl] = jnp.maximum(
                                hr[u][e, sl] + hc[u & 1][e, sl], 0.0)
                    return 0

                lax.fori_loop(0, K // 4, compute, 0)
                fire_scatter(u)

        return 0

    lax.fori_loop(0, LOOPN // 4, quad, 0)

    @pl.when(valid(LOOPN - 2))
    def _():
        wait_scatter((LOOPN - 2) & 3)

    @pl.when(valid(LOOPN - 1))
    def _():
        wait_scatter((LOOPN - 1) & 3)

    plsc.subcore_barrier()

    rows = pl.ds(sid * STRIPE, STRIPE)
    pltpu.sync_copy(acc_sh.at[rows, :], acc_out.at[cid, rows, :])


@functools.lru_cache(maxsize=1)
def _sc_kernels():
    mesh = plsc.VectorSubcoreMesh(core_axis_name="c", subcore_axis_name="s")
    params = pltpu.CompilerParams(use_tc_tiling_on_sc=False)
    pool_values = pl.kernel(
        _sc_pool_values_body,
        out_type=[
            jax.ShapeDtypeStruct((NC, NP, C_IN), jnp.float32),
            jax.ShapeDtypeStruct((NC, NP), jnp.float32),
        ],
        mesh=mesh,
        scratch_types=(
            [pltpu.VMEM((K,), jnp.int32)] * 4
            + [pltpu.VMEM((K, C_IN), jnp.float32)] * 4
            + [pltpu.VMEM((K,), jnp.float32),
               pltpu.VMEM((K, C_IN), jnp.float32),
               pltpu.VMEM((K,), jnp.float32)]
            + [pltpu.SemaphoreType.DMA] * 8
            + [pltpu.MemorySpace.VMEM_SHARED((NP, C_IN), jnp.float32),
               pltpu.MemorySpace.VMEM_SHARED((NP,), jnp.float32)]
        ),
        compiler_params=params,
    )
    edge_stage = pl.kernel(
        _sc_edge_stage_body,
        out_type=jax.ShapeDtypeStruct((NC, NP, EMB), jnp.float32),
        mesh=mesh,
        scratch_types=(
            [pltpu.VMEM((2, K), jnp.int32)] * 4
            + [pltpu.VMEM((K, EMB), jnp.float32)] * 6
            + [pltpu.VMEM((64, EMB), jnp.float32)]
            + [pltpu.SemaphoreType.DMA] * 12
            + [pltpu.MemorySpace.VMEM_SHARED((NP, EMB), jnp.float32)]
        ),
        compiler_params=params,
    )
    return pool_values, edge_stage


BLK = 6400
NB = NP // BLK


def _mean_of(acc_r, acc_c, cnt_r, cnt_c):
    cr = cnt_r[0]
    cc = cnt_c[0]
    mean = acc_r[0] / jnp.maximum(cr, 1.0) + acc_c[0] / jnp.maximum(cc, 1.0)
    gate = (cr > 0.0).astype(jnp.float32) + (cc > 0.0).astype(jnp.float32)
    return mean, gate


def _pre_body(with_emb):
    def body(*refs):
        if with_emb:
            acc_r, acc_c, cnt_r, cnt_c, emb0, wp, bp, x_out, st_out = refs
        else:
            acc_r, acc_c, cnt_r, cnt_c, wp, bp, x_out, st_out = refs
        mean, gate = _mean_of(acc_r, acc_c, cnt_r, cnt_c)
        pooled = (jnp.dot(mean, wp[:], preferred_element_type=jnp.float32)
                  + gate * bp[:])
        if with_emb:
            rid = (pl.program_id(0) * BLK
                   + lax.broadcasted_iota(jnp.int32, (BLK, 1), 0))
            pooled = pooled + jnp.where(rid < N_NODES, emb0[:], 0.0)
        x = jnp.maximum(pooled, 0.0)
        x_out[:] = x
        part = jnp.concatenate(
            [jnp.sum(x, axis=0, keepdims=True),
             jnp.sum(x * x, axis=0, keepdims=True)], axis=0)

        @pl.when(pl.program_id(0) == 0)
        def _():
            st_out[:] = part

        @pl.when(pl.program_id(0) != 0)
        def _():
            st_out[:] = st_out[:] + part

    return body


def _post_body(x, st, wb, bb, out):
    m = st[0:1, :] / N_NODES
    v = st[1:2, :] / N_NODES - m * m
    y = (x[:] - m) * lax.rsqrt(v + 1e-5)
    out[:] = jnp.dot(y, wb[:], preferred_element_type=jnp.float32) + bb[:]


def _densef_body(acc_r, acc_c, cnt_r, cnt_c, wp, bp, out):
    mean, gate = _mean_of(acc_r, acc_c, cnt_r, cnt_c)
    out[:] = jnp.dot(mean, wp[:], preferred_element_type=jnp.float32) + gate * bp[:]


def _half(core, c):
    return pl.BlockSpec((1, BLK, c), lambda i, _c=core: (_c, i, 0))


def _rows(c):
    return pl.BlockSpec((BLK, c), lambda i: (i, 0))


def _full(r, c):
    return pl.BlockSpec((r, c), lambda i: (0, 0))


def _dense_pre(acc, cnt3, emb0, wp, bp):
    cin = acc.shape[2]
    with_emb = emb0 is not None
    specs = [_half(0, cin), _half(1, cin), _half(0, 1), _half(1, 1)]
    args = [acc, acc, cnt3, cnt3]
    if with_emb:
        specs.append(_rows(EMB))
        args.append(emb0)
    specs += [_full(cin, EMB), _full(1, EMB)]
    args += [wp, bp]
    return pl.pallas_call(
        _pre_body(with_emb),
        grid=(NB,),
        in_specs=specs,
        out_specs=[_rows(EMB), _full(2, EMB)],
        out_shape=[jax.ShapeDtypeStruct((NP, EMB), jnp.float32),
                   jax.ShapeDtypeStruct((2, EMB), jnp.float32)],
    )(*args)


def _dense_post(x, st, wb, bb):
    return pl.pallas_call(
        _post_body,
        grid=(NB,),
        in_specs=[_rows(EMB), _full(2, EMB), _full(EMB, EMB), _full(1, EMB)],
        out_specs=_rows(EMB),
        out_shape=jax.ShapeDtypeStruct((NP, EMB), jnp.float32),
    )(x, st, wb, bb)


def _densef(acc, cnt3, wp, bp):
    return pl.pallas_call(
        _densef_body,
        grid=(NB,),
        in_specs=[_half(0, EMB), _half(1, EMB), _half(0, 1), _half(1, 1),
                  _full(EMB, 1), _full(1, 1)],
        out_specs=_rows(1),
        out_shape=jax.ShapeDtypeStruct((NP, 1), jnp.float32),
    )(acc, acc, cnt3, cnt3, wp, bp)


def kernel(data_values, data_embedding, edge_index, W_pool0, b_pool0,
           W_pool1, b_pool1, W_pool2, b_pool2, W_bc0, b_bc0, W_bc1, b_bc1):
    ei = edge_index.astype(jnp.int32)
    _sc_pool_values, _sc_edge_stage = _sc_kernels()

    acc0, cnt = _sc_pool_values(data_values, ei)
    cnt3 = cnt.reshape(NC, NP, 1)

    x0, st0 = _dense_pre(acc0, cnt3, data_embedding,
                         W_pool0, b_pool0.reshape(1, EMB))
    h0 = _dense_post(x0, st0, W_bc0, b_bc0.reshape(1, EMB))
    acc1 = _sc_edge_stage(h0, ei)
    x1, st1 = _dense_pre(acc1, cnt3, None, W_pool1, b_pool1.reshape(1, EMB))
    h1 = _dense_post(x1, st1, W_bc1, b_bc1.reshape(1, EMB))
    acc2 = _sc_edge_stage(h1, ei)
    out = _densef(acc2, cnt3, W_pool2, b_pool2.reshape(1, 1))
    return out[:N_NODES]
